# baseline (speedup 1.0000x reference)
import functools

import jax
import jax.numpy as jnp
from jax import lax
from jax.experimental import pallas as pl
from jax.experimental.pallas import tpu as pltpu
from jax.experimental.pallas import tpu_sc as plsc

_NC = 2
_NS = 16
_NW = _NC * _NS

_CHUNK = 512


def _embed_lookup(idx_flat, table):
    (n,) = idx_flat.shape
    _, d = table.shape
    b_per_w = n // _NW
    n_steps = b_per_w // _CHUNK
    mesh = plsc.VectorSubcoreMesh(core_axis_name="c", subcore_axis_name="s")

    @functools.partial(
        pl.kernel,
        mesh=mesh,
        out_type=jax.ShapeDtypeStruct((n, d), jnp.float32),
        scratch_types=[
            pltpu.VMEM((_CHUNK,), jnp.int32),
            pltpu.VMEM((_CHUNK, d), jnp.float32),
            pltpu.SemaphoreType.DMA,
        ],
        compiler_params=pltpu.CompilerParams(use_tc_tiling_on_sc=False),
    )
    def k(idx_hbm, tab_hbm, out_hbm, idx_v, rows_v, sem):
        wid = lax.axis_index("s") * _NC + lax.axis_index("c")
        base = wid * b_per_w

        def body(i, carry):
            off = base + i * _CHUNK
            pltpu.sync_copy(idx_hbm.at[pl.ds(off, _CHUNK)], idx_v)
            pltpu.async_copy(tab_hbm.at[idx_v], rows_v, sem).wait()
            pltpu.sync_copy(rows_v, out_hbm.at[pl.ds(off, _CHUNK)])
            return carry

        lax.fori_loop(0, n_steps, body, 0)

    return k(idx_flat, table)


def kernel(sentence, word_embeds):
    b, s = sentence.shape
    d = word_embeds.shape[1]
    idx = sentence.reshape(-1).astype(jnp.int32)
    out = _embed_lookup(idx, word_embeds)
    return out.reshape(b, s, d)


# speedup vs baseline: 1.0451x; 1.0451x over previous
import functools

import jax
import jax.numpy as jnp
from jax import lax
from jax.experimental import pallas as pl
from jax.experimental.pallas import tpu as pltpu
from jax.experimental.pallas import tpu_sc as plsc

_NC = 2
_NS = 16
_NW = _NC * _NS

_CHUNK = 400
_NBUF = 4


def _embed_lookup(idx3, table):
    nw, n_steps, chunk = idx3.shape
    _, d = table.shape
    b_per_w = n_steps * chunk
    n = nw * b_per_w
    n_groups = n_steps // _NBUF
    mesh = plsc.VectorSubcoreMesh(core_axis_name="c", subcore_axis_name="s")

    @functools.partial(
        pl.kernel,
        mesh=mesh,
        out_type=jax.ShapeDtypeStruct((n, d), jnp.float32),
        scratch_types=[
            pltpu.VMEM((n_steps, chunk), jnp.int32),
            pltpu.VMEM((_NBUF, chunk, d), jnp.float32),
            pltpu.SemaphoreType.DMA((_NBUF,)),
            pltpu.SemaphoreType.DMA((_NBUF,)),
        ],
        compiler_params=pltpu.CompilerParams(use_tc_tiling_on_sc=False),
    )
    def k(idx_hbm, tab_hbm, out_hbm, idx_all, rows, gsem, ssem):
        wid = lax.axis_index("s") * _NC + lax.axis_index("c")
        base = wid * b_per_w

        def gather_start(b, i):
            pltpu.async_copy(tab_hbm.at[idx_all.at[i]], rows.at[b], gsem.at[b])

        def gather_wait(b, i):
            pltpu.make_async_copy(
                tab_hbm.at[idx_all.at[i]], rows.at[b], gsem.at[b]).wait()

        def store_start(b, i):
            pltpu.async_copy(
                rows.at[b], out_hbm.at[pl.ds(base + i * chunk, chunk)],
                ssem.at[b])

        def store_wait(b, i):
            pltpu.make_async_copy(
                rows.at[b], out_hbm.at[pl.ds(base + i * chunk, chunk)],
                ssem.at[b]).wait()

        pltpu.sync_copy(idx_hbm.at[wid], idx_all)

        for b in range(_NBUF):
            gather_start(b, b)
        gather_wait(0, 0)
        store_start(0, 0)

        def group(g, carry):
            for b in range(_NBUF):
                i = g * _NBUF + b
                store_wait(b, i - _NBUF)
                gather_start(b, i)
                kk = i - (_NBUF - 1)
                bk = (b + 1) % _NBUF
                gather_wait(bk, kk)
                store_start(bk, kk)
            return carry

        lax.fori_loop(1, n_groups, group, 0, unroll=False)

        for kk in range(n_steps - (_NBUF - 1), n_steps):
            gather_wait(kk % _NBUF, kk)
            store_start(kk % _NBUF, kk)
        for b in range(_NBUF):
            store_wait(b, n_steps - _NBUF + b)

    return k(idx3, table)


def kernel(sentence, word_embeds):
    b, s = sentence.shape
    d = word_embeds.shape[1]
    n = b * s
    b_per_w = n // _NW
    n_steps = b_per_w // _CHUNK
    idx3 = sentence.reshape(_NW, n_steps, _CHUNK).astype(jnp.int32)
    out = _embed_lookup(idx3, word_embeds)
    return out.reshape(b, s, d)
